# Initial kernel scaffold; baseline (speedup 1.0000x reference)
import jax
import jax.numpy as jnp
from jax.experimental import pallas as pl


def kernel(x, edge_index, W_in, b_in, g_in, be_in, Wc1_0, bc1_0, Wc2_0, bc2_0, g1_0, be1_0, g2_0, be2_0, Wc1_1, bc1_1, Wc2_1, bc2_1, g1_1, be1_1, g2_1, be2_1, Ws_1, bs_1, Wc1_2, bc1_2, Wc2_2, bc2_2, g1_2, be1_2, g2_2, be2_2, Ws_2, bs_2, We1, be1, ge1, bee1, We2, be2, ge2, bee2, c_W1, c_b1, c_g1, c_bb1, c_W2, c_b2, c_g2, c_bb2, c_W3, c_b3, r_W1, r_b1, r_g1, r_bb1, r_W2, r_b2, r_g2, r_bb2, r_W3, r_b3):
    raise NotImplementedError("write your pallas kernel here")


# speedup vs baseline: 5.3918x; 5.3918x over previous
import functools

import jax
import jax.numpy as jnp
from jax import lax
from jax.experimental import pallas as pl
from jax.experimental.pallas import tpu as pltpu
from jax.experimental.pallas import tpu_sc as plsc

N = 10000
E = 320000
NPAD = 10240
NSLAB = 10112
FR = NSLAB // 16
EPAD = 327680
IB = 40
DUMMY_DST = 10008
TN = 256
NT = NPAD // TN
EPS = 1e-5


def _sc_mesh():
    return plsc.VectorSubcoreMesh(core_axis_name="c", subcore_axis_name="s")


@functools.lru_cache(maxsize=None)
def _get_sc_degree():
    @functools.partial(
        pl.kernel,
        out_type=jax.ShapeDtypeStruct((2, NPAD, 16), jnp.float32),
        mesh=_sc_mesh(),
        scratch_types=[
            pltpu.VMEM((80, 128), jnp.int32),
            pltpu.VMEM((128, 16), jnp.float32),
            pltpu.VMEM_SHARED((NSLAB, 16), jnp.float32),
        ],
    )
    def _sc_degree(dstp, ones_hbm, zeros_hbm, out, idx_d, ones_v, slab):
        cc = lax.axis_index("c")
        sid = lax.axis_index("s")
        wid = cc * 16 + sid
        pltpu.sync_copy(dstp.at[pl.ds(wid * 80, 80)], idx_d)
        pltpu.sync_copy(ones_hbm, ones_v)
        pltpu.sync_copy(zeros_hbm.at[pl.ds(0, FR)], slab.at[pl.ds(sid * FR, FR)])
        plsc.subcore_barrier()

        def body(j, carry):
            pltpu.sync_copy(ones_v, slab.at[idx_d.at[j]], add=True)
            return carry

        lax.fori_loop(0, 80, body, 0)
        plsc.subcore_barrier()
        pltpu.sync_copy(slab.at[pl.ds(sid * FR, FR)],
                        out.at[cc].at[pl.ds(sid * FR, FR)])

    return _sc_degree


@functools.lru_cache(maxsize=None)
def _make_sc_agg(C):
    n_out = 2 if C == 1 else C
    chunks_per_core = 1 if C == 1 else C // 2
    steps = (EPAD // 128) // (32 if C == 1 else 16)

    nblk = steps // IB

    @functools.partial(
        pl.kernel,
        out_type=jax.ShapeDtypeStruct((n_out, NPAD, 128), jnp.float32),
        mesh=_sc_mesh(),
        scratch_types=[
            pltpu.VMEM((IB, 128), jnp.int32),
            pltpu.VMEM((IB, 128), jnp.int32),
            pltpu.VMEM((2, 128, 128), jnp.float32),
            pltpu.VMEM_SHARED((NSLAB, 128), jnp.float32),
            pltpu.SemaphoreType.DMA,
            pltpu.SemaphoreType.DMA,
        ],
    )
    def agg(table, srcp, dstp, zeros_hbm, out, idx_s, idx_d, rows, slab,
            sem0, sem1):
        cc = lax.axis_index("c")
        sid = lax.axis_index("s")
        row0 = (cc * 16 + sid) * steps if C == 1 else sid * steps
        sems = (sem0, sem1)

        for k in range(chunks_per_core):
            chunk = 0 if C == 1 else cc * chunks_per_core + k
            outidx = cc if C == 1 else chunk
            tbl = table.at[chunk]
            drain = tbl.at[pl.ds(0, 128)]
            pltpu.sync_copy(zeros_hbm.at[pl.ds(0, FR)],
                            slab.at[pl.ds(sid * FR, FR)])
            plsc.subcore_barrier()

            def blk_body(bi, carry):
                pltpu.sync_copy(srcp.at[pl.ds(row0 + bi * IB, IB)], idx_s)
                pltpu.sync_copy(dstp.at[pl.ds(row0 + bi * IB, IB)], idx_d)
                pltpu.async_copy(tbl.at[idx_s.at[0]], rows.at[0], sem0)
                pltpu.async_copy(tbl.at[idx_s.at[1]], rows.at[1], sem1)

                def body(i, c2):
                    for b in range(2):
                        j = 2 * i + b
                        pltpu.make_async_copy(drain, rows.at[b],
                                              sems[b]).wait()
                        pltpu.sync_copy(rows.at[b], slab.at[idx_d.at[j]],
                                        add=True)

                        @pl.when(j + 2 < IB)
                        def _():
                            pltpu.async_copy(tbl.at[idx_s.at[j + 2]],
                                             rows.at[b], sems[b])
                    return c2

                lax.fori_loop(0, IB // 2, body, 0)
                return carry

            lax.fori_loop(0, nblk, blk_body, 0)
            plsc.subcore_barrier()
            pltpu.sync_copy(slab.at[pl.ds(sid * FR, FR)],
                            out.at[outidx].at[pl.ds(sid * FR, FR)])
            if k + 1 < chunks_per_core:
                plsc.subcore_barrier()

    return agg


def _mm(a, w):
    return lax.dot_general(a, w, (((1,), (1,)), ((), ())),
                           precision=lax.Precision.HIGHEST,
                           preferred_element_type=jnp.float32)


def _ln(x, g, b):
    mu = jnp.mean(x, axis=-1, keepdims=True)
    var = jnp.mean((x - mu) ** 2, axis=-1, keepdims=True)
    return (x - mu) / jnp.sqrt(var + EPS) * g + b


def _gelu(x):
    return 0.5 * x * (1.0 + lax.erf(x * 0.7071067811865476))


def _row_spec(d):
    return pl.BlockSpec((1, d), lambda i: (0, 0))


def _full2_spec(shape):
    return pl.BlockSpec(shape, lambda i: (0, 0))


def _tile_spec(d):
    return pl.BlockSpec((TN, d), lambda i: (i, 0))


def _chunk_spec(c):
    return pl.BlockSpec((c, TN, 128), lambda i: (0, i, 0))


def _ln_chunks(ys, g_ref, be_ref, width):
    m = ys[0].sum(axis=1, keepdims=True)
    for y in ys[1:]:
        m = m + y.sum(axis=1, keepdims=True)
    m = m / width
    v = ((ys[0] - m) ** 2).sum(axis=1, keepdims=True)
    for y in ys[1:]:
        v = v + ((y - m) ** 2).sum(axis=1, keepdims=True)
    v = v / width
    inv = 1.0 / jnp.sqrt(v + EPS)
    outs = []
    for c, y in enumerate(ys):
        g = g_ref[:, c * 128:(c + 1) * 128]
        b = be_ref[:, c * 128:(c + 1) * 128]
        outs.append((y - m) * inv * g + b)
    return outs


def _dinv_body(deg_ref, out_ref):
    d = 1.0 + deg_ref[0, :, 0:1] + deg_ref[1, :, 0:1]
    out_ref[...] = lax.rsqrt(d)


def _dinv(deg):
    return pl.pallas_call(
        _dinv_body,
        grid=(NT,),
        in_specs=[pl.BlockSpec((2, TN, 16), lambda i: (0, i, 0))],
        out_specs=_tile_spec(1),
        out_shape=jax.ShapeDtypeStruct((NPAD, 1), jnp.float32),
    )(deg)


def _s0a_body(x_ref, win_ref, bin_ref, gin_ref, bein_ref, h_ref):
    h = _mm(x_ref[...], win_ref[...]) + bin_ref[...]
    h_ref[...] = _gelu(_ln(h, gin_ref[...], bein_ref[...]))


def _s0a(xp, W_in, b_in, g_in, be_in):
    return pl.pallas_call(
        _s0a_body,
        grid=(NT,),
        in_specs=[
            _tile_spec(128),
            _full2_spec((512, 128)),
            _row_spec(512), _row_spec(512), _row_spec(512),
        ],
        out_specs=_tile_spec(512),
        out_shape=jax.ShapeDtypeStruct((NPAD, 512), jnp.float32),
    )(xp, W_in, b_in, g_in, be_in)


def _s0b_body(h_ref, wc1_ref, dinv_ref, tbl_ref):
    h = h_ref[...]
    dv = dinv_ref[...]
    for d in range(4):
        w = wc1_ref[...][d * 128:(d + 1) * 128, :]
        tbl_ref[d] = _mm(h, w) * dv


def _s0b(h0, Wc1, dinv):
    return pl.pallas_call(
        _s0b_body,
        grid=(NT,),
        in_specs=[
            _tile_spec(512),
            _full2_spec((512, 512)),
            _tile_spec(1),
        ],
        out_specs=_chunk_spec(4),
        out_shape=jax.ShapeDtypeStruct((4, NPAD, 128), jnp.float32),
    )(h0, Wc1, dinv)


def _make_t2(C, C2, partial_in):
    width = C * 128
    n_in = 2 if partial_in else C

    def body(agg_ref, tbl_ref, dinv_ref, b_ref, g_ref, be_ref, w2_ref,
             out_ref):
        dv = dinv_ref[...]
        ys = []
        for c in range(C):
            if partial_in:
                a = agg_ref[0] + agg_ref[1]
            else:
                a = agg_ref[c]
            b = b_ref[:, c * 128:(c + 1) * 128]
            ys.append(dv * (a + tbl_ref[c]) + b)
        zs = [_gelu(z) for z in _ln_chunks(ys, g_ref, be_ref, width)]
        w2 = w2_ref[...]
        for d in range(C2):
            acc = None
            for c in range(C):
                wblk = w2[d * 128:(d + 1) * 128, c * 128:(c + 1) * 128]
                t = _mm(zs[c], wblk)
                acc = t if acc is None else acc + t
            out_ref[d] = acc * dv

    def call(agg, tbl, dinv, b, g, be, W2):
        return pl.pallas_call(
            body,
            grid=(NT,),
            in_specs=[
                _chunk_spec(n_in), _chunk_spec(C), _tile_spec(1),
                _row_spec(width), _row_spec(width), _row_spec(width),
                _full2_spec((C2 * 128, width)),
            ],
            out_specs=_chunk_spec(C2),
            out_shape=jax.ShapeDtypeStruct((C2, NPAD, 128), jnp.float32),
        )(agg, tbl, dinv, b, g, be, W2)

    return call


def _make_t3(C, C_next, skip_width, partial_in):
    width = C * 128
    n_in = 2 if partial_in else C
    proj = skip_width > 0

    def body(*refs):
        it = iter(refs)
        agg_ref = next(it); tbl_ref = next(it); dinv_ref = next(it)
        b_ref = next(it); g_ref = next(it); be_ref = next(it)
        hprev_ref = next(it)
        if proj:
            ws_ref = next(it); bs_ref = next(it)
        wn_ref = next(it)
        hout_ref = next(it); tbl_out_ref = next(it)

        dv = dinv_ref[...]
        ys = []
        for c in range(C):
            a = (agg_ref[0] + agg_ref[1]) if partial_in else agg_ref[c]
            b = b_ref[:, c * 128:(c + 1) * 128]
            ys.append(dv * (a + tbl_ref[c]) + b)
        os_ = _ln_chunks(ys, g_ref, be_ref, width)
        hp = hprev_ref[...]
        hs = []
        for c in range(C):
            if proj:
                wblk = ws_ref[...][c * 128:(c + 1) * 128, :]
                skip = _mm(hp, wblk) + bs_ref[:, c * 128:(c + 1) * 128]
            else:
                skip = hp[:, c * 128:(c + 1) * 128]
            h = _gelu(os_[c] + skip)
            hs.append(h)
            hout_ref[:, c * 128:(c + 1) * 128] = h
        wn = wn_ref[...]
        for d in range(C_next):
            acc = None
            for c in range(C):
                wblk = wn[d * 128:(d + 1) * 128, c * 128:(c + 1) * 128]
                t = _mm(hs[c], wblk)
                acc = t if acc is None else acc + t
            tbl_out_ref[d] = acc * dv

    def call(agg, tbl, dinv, b, g, be, hprev, ws, bs, Wn):
        in_specs = [
            _chunk_spec(n_in), _chunk_spec(C), _tile_spec(1),
            _row_spec(width), _row_spec(width), _row_spec(width),
            _tile_spec(skip_width if proj else width),
        ]
        args = [agg, tbl, dinv, b, g, be, hprev]
        if proj:
            in_specs += [_full2_spec((width, skip_width)), _row_spec(width)]
            args += [ws, bs]
        in_specs.append(_full2_spec((C_next * 128, width)))
        args.append(Wn)
        return pl.pallas_call(
            body,
            grid=(NT,),
            in_specs=in_specs,
            out_specs=[_tile_spec(width), _chunk_spec(C_next)],
            out_shape=[
                jax.ShapeDtypeStruct((NPAD, width), jnp.float32),
                jax.ShapeDtypeStruct((C_next, NPAD, 128), jnp.float32),
            ],
        )(*args)

    return call


def _t3_final_body(agg_ref, tbl_ref, dinv_ref, b_ref, g_ref, be_ref,
                   hprev_ref, ws_ref, bs_ref,
                   we1_ref, be1_ref, ge1_ref, bee1_ref,
                   we2_ref, be2_ref, ge2_ref, bee2_ref,
                   pooled_ref):
    i = pl.program_id(0)
    dv = dinv_ref[...]
    y = dv * (agg_ref[0] + agg_ref[1] + tbl_ref[0]) + b_ref[...]
    o = _ln(y, g_ref[...], be_ref[...])
    skip = _mm(hprev_ref[...], ws_ref[...]) + bs_ref[...]
    h3 = _gelu(o + skip)
    e = _mm(h3, we1_ref[...]) + be1_ref[...]
    e = _gelu(_ln(e, ge1_ref[...], bee1_ref[...]))
    node = _mm(e, we2_ref[...]) + be2_ref[...]
    node = _ln(node, ge2_ref[...], bee2_ref[...])
    ridx = i * TN + lax.broadcasted_iota(jnp.int32, (TN, 1), 0)
    mask = ridx < N
    nsum = jnp.sum(jnp.where(mask, node, 0.0), axis=0, keepdims=True)
    nmax = jnp.max(jnp.where(mask, node, -jnp.inf), axis=0, keepdims=True)

    @pl.when(i == 0)
    def _():
        pooled_ref[0:1] = nsum
        pooled_ref[1:2] = nmax

    @pl.when(i > 0)
    def _():
        pooled_ref[0:1] += nsum
        pooled_ref[1:2] = jnp.maximum(pooled_ref[1:2], nmax)


def _t3_final(agg, tbl, dinv, b, g, be, hprev, ws, bs,
              We1, be1, ge1, bee1, We2, be2, ge2, bee2):
    return pl.pallas_call(
        _t3_final_body,
        grid=(NT,),
        in_specs=[
            _chunk_spec(2), _chunk_spec(1), _tile_spec(1),
            _row_spec(128), _row_spec(128), _row_spec(128),
            _tile_spec(256),
            _full2_spec((128, 256)), _row_spec(128),
            _full2_spec((256, 128)), _row_spec(256), _row_spec(256),
            _row_spec(256),
            _full2_spec((128, 256)), _row_spec(128), _row_spec(128),
            _row_spec(128),
        ],
        out_specs=pl.BlockSpec((8, 128), lambda i: (0, 0)),
        out_shape=jax.ShapeDtypeStruct((8, 128), jnp.float32),
    )(agg, tbl, dinv, b, g, be, hprev, ws, bs,
      We1, be1, ge1, bee1, We2, be2, ge2, bee2)


def _head_body(pooled_ref,
               cw1_ref, cb1_ref, cg1_ref, cbb1_ref, cw2_ref, cb2_ref,
               cg2_ref, cbb2_ref, cw3_ref, cb3_ref,
               rw1_ref, rb1_ref, rg1_ref, rbb1_ref, rw2_ref, rb2_ref,
               rg2_ref, rbb2_ref, rw3_ref, rb3_ref,
               cls_ref, reg_ref):
    s = pooled_ref[0:1]
    mx = pooled_ref[1:2]
    g = jnp.concatenate([s / float(N), mx, s], axis=1)

    def mlp(w1, b1, g1, bb1, w2, b2, g2, bb2, w3, b3):
        z = _mm(g, w1[...]) + b1[...]
        z = _gelu(_ln(z, g1[...], bb1[...]))
        z = _mm(z, w2[...]) + b2[...]
        z = _gelu(_ln(z, g2[...], bb2[...]))
        return _mm(z, w3[...]) + b3[...]

    cls_ref[...] = mlp(cw1_ref, cb1_ref, cg1_ref, cbb1_ref, cw2_ref,
                       cb2_ref, cg2_ref, cbb2_ref, cw3_ref, cb3_ref)
    reg_ref[...] = jax.nn.sigmoid(
        mlp(rw1_ref, rb1_ref, rg1_ref, rbb1_ref, rw2_ref, rb2_ref,
            rg2_ref, rbb2_ref, rw3_ref, rb3_ref))


def _pad_head_w3(w3, b3):
    od = w3.shape[0]
    return (jnp.pad(w3, ((0, 128 - od), (0, 0))),
            jnp.pad(b3.reshape(1, -1), ((0, 0), (0, 128 - od))))


def _head(pooled, cparams, rparams):
    def spec(a):
        return pl.BlockSpec(a.shape, lambda: tuple(0 for _ in a.shape))

    args = [pooled] + list(cparams) + list(rparams)
    return pl.pallas_call(
        _head_body,
        in_specs=[spec(a) for a in args],
        out_specs=[pl.BlockSpec((1, 128), lambda: (0, 0)),
                   pl.BlockSpec((1, 128), lambda: (0, 0))],
        out_shape=[jax.ShapeDtypeStruct((1, 128), jnp.float32),
                   jax.ShapeDtypeStruct((1, 128), jnp.float32)],
    )(*args)


def kernel(x, edge_index, W_in, b_in, g_in, be_in,
           Wc1_0, bc1_0, Wc2_0, bc2_0, g1_0, be1_0, g2_0, be2_0,
           Wc1_1, bc1_1, Wc2_1, bc2_1, g1_1, be1_1, g2_1, be2_1, Ws_1, bs_1,
           Wc1_2, bc1_2, Wc2_2, bc2_2, g1_2, be1_2, g2_2, be2_2, Ws_2, bs_2,
           We1, be1, ge1, bee1, We2, be2, ge2, bee2,
           c_W1, c_b1, c_g1, c_bb1, c_W2, c_b2, c_g2, c_bb2, c_W3, c_b3,
           r_W1, r_b1, r_g1, r_bb1, r_W2, r_b2, r_g2, r_bb2, r_W3, r_b3):
    r = lambda a: a.reshape(1, -1)
    xp = jnp.pad(x, ((0, NPAD - N), (0, 0)))
    src = edge_index[0]
    dst = edge_index[1]
    srcp = jnp.concatenate(
        [src, jnp.zeros((EPAD - E,), jnp.int32)]).reshape(EPAD // 128, 128)
    dstp = jnp.concatenate(
        [dst, jnp.full((EPAD - E,), DUMMY_DST, jnp.int32)]
    ).reshape(EPAD // 128, 128)
    zeros16 = jnp.zeros((640, 16), jnp.float32)
    ones16 = jnp.ones((128, 16), jnp.float32)
    zeros128 = jnp.zeros((640, 128), jnp.float32)

    deg = _get_sc_degree()(dstp, ones16, zeros16)
    h0 = _s0a(xp, W_in, r(b_in), r(g_in), r(be_in))
    dinv = _dinv(deg)
    tbl = _s0b(h0, Wc1_0, dinv)

    t2_0 = _make_t2(4, 4, False)
    t3_0 = _make_t3(4, 2, 0, False)
    t2_1 = _make_t2(2, 2, False)
    t3_1 = _make_t3(2, 1, 512, False)
    t2_2 = _make_t2(1, 1, True)

    agg = _make_sc_agg(4)(tbl, srcp, dstp, zeros128)
    tbl = t2_0(agg, tbl, dinv, r(bc1_0), r(g1_0), r(be1_0), Wc2_0)
    agg = _make_sc_agg(4)(tbl, srcp, dstp, zeros128)
    h1, tbl = t3_0(agg, tbl, dinv, r(bc2_0), r(g2_0), r(be2_0), h0,
                   None, None, Wc1_1)

    agg = _make_sc_agg(2)(tbl, srcp, dstp, zeros128)
    tbl = t2_1(agg, tbl, dinv, r(bc1_1), r(g1_1), r(be1_1), Wc2_1)
    agg = _make_sc_agg(2)(tbl, srcp, dstp, zeros128)
    h2, tbl = t3_1(agg, tbl, dinv, r(bc2_1), r(g2_1), r(be2_1), h1,
                   Ws_1, r(bs_1), Wc1_2)

    agg = _make_sc_agg(1)(tbl, srcp, dstp, zeros128)
    tbl = t2_2(agg, tbl, dinv, r(bc1_2), r(g1_2), r(be1_2), Wc2_2)
    agg = _make_sc_agg(1)(tbl, srcp, dstp, zeros128)
    pooled = _t3_final(agg, tbl, dinv, r(bc2_2), r(g2_2), r(be2_2), h2,
                       Ws_2, r(bs_2),
                       We1, r(be1), r(ge1), r(bee1),
                       We2, r(be2), r(ge2), r(bee2))

    cls, reg = _head(
        pooled,
        (c_W1, r(c_b1), r(c_g1), r(c_bb1), c_W2, r(c_b2), r(c_g2),
         r(c_bb2), *_pad_head_w3(c_W3, c_b3)),
        (r_W1, r(r_b1), r(r_g1), r(r_bb1), r_W2, r(r_b2), r(r_g2),
         r(r_bb2), *_pad_head_w3(r_W3, r_b3)))
    return (cls[:, :5], reg[:, :1])


# speedup vs baseline: 5.4822x; 1.0168x over previous
import functools

import jax
import jax.numpy as jnp
from jax import lax
from jax.experimental import pallas as pl
from jax.experimental.pallas import tpu as pltpu
from jax.experimental.pallas import tpu_sc as plsc

N = 10000
E = 320000
NPAD = 10240
NSLAB = 10112
FR = NSLAB // 16
EPAD = 327680
IB = 40
DUMMY_DST = 10008
TN = 512
NT = NPAD // TN
EPS = 1e-5


def _sc_mesh():
    return plsc.VectorSubcoreMesh(core_axis_name="c", subcore_axis_name="s")


@functools.lru_cache(maxsize=None)
def _get_sc_degree():
    @functools.partial(
        pl.kernel,
        out_type=jax.ShapeDtypeStruct((2, NPAD, 16), jnp.float32),
        mesh=_sc_mesh(),
        scratch_types=[
            pltpu.VMEM((80, 128), jnp.int32),
            pltpu.VMEM((128, 16), jnp.float32),
            pltpu.VMEM_SHARED((NSLAB, 16), jnp.float32),
        ],
    )
    def _sc_degree(dstp, ones_hbm, zeros_hbm, out, idx_d, ones_v, slab):
        cc = lax.axis_index("c")
        sid = lax.axis_index("s")
        wid = cc * 16 + sid
        pltpu.sync_copy(dstp.at[pl.ds(wid * 80, 80)], idx_d)
        pltpu.sync_copy(ones_hbm, ones_v)
        pltpu.sync_copy(zeros_hbm.at[pl.ds(0, FR)], slab.at[pl.ds(sid * FR, FR)])
        plsc.subcore_barrier()

        def body(j, carry):
            pltpu.sync_copy(ones_v, slab.at[idx_d.at[j]], add=True)
            return carry

        lax.fori_loop(0, 80, body, 0)
        plsc.subcore_barrier()
        pltpu.sync_copy(slab.at[pl.ds(sid * FR, FR)],
                        out.at[cc].at[pl.ds(sid * FR, FR)])

    return _sc_degree


@functools.lru_cache(maxsize=None)
def _make_sc_agg(C):
    n_out = 2 if C == 1 else C
    chunks_per_core = 1 if C == 1 else C // 2
    steps = (EPAD // 128) // (32 if C == 1 else 16)

    nblk = steps // IB

    @functools.partial(
        pl.kernel,
        out_type=jax.ShapeDtypeStruct((n_out, NPAD, 128), jnp.float32),
        mesh=_sc_mesh(),
        scratch_types=[
            pltpu.VMEM((IB, 128), jnp.int32),
            pltpu.VMEM((IB, 128), jnp.int32),
            pltpu.VMEM((2, 128, 128), jnp.float32),
            pltpu.VMEM_SHARED((NSLAB, 128), jnp.float32),
            pltpu.SemaphoreType.DMA,
            pltpu.SemaphoreType.DMA,
        ],
    )
    def agg(table, srcp, dstp, zeros_hbm, out, idx_s, idx_d, rows, slab,
            sem0, sem1):
        cc = lax.axis_index("c")
        sid = lax.axis_index("s")
        row0 = (cc * 16 + sid) * steps if C == 1 else sid * steps
        sems = (sem0, sem1)

        for k in range(chunks_per_core):
            chunk = 0 if C == 1 else cc * chunks_per_core + k
            outidx = cc if C == 1 else chunk
            tbl = table.at[chunk]
            drain = tbl.at[pl.ds(0, 128)]
            pltpu.sync_copy(zeros_hbm.at[pl.ds(0, FR)],
                            slab.at[pl.ds(sid * FR, FR)])
            plsc.subcore_barrier()

            def blk_body(bi, carry):
                pltpu.sync_copy(srcp.at[pl.ds(row0 + bi * IB, IB)], idx_s)
                pltpu.sync_copy(dstp.at[pl.ds(row0 + bi * IB, IB)], idx_d)
                pltpu.async_copy(tbl.at[idx_s.at[0]], rows.at[0], sem0)
                pltpu.async_copy(tbl.at[idx_s.at[1]], rows.at[1], sem1)

                def body(i, c2):
                    for b in range(2):
                        j = 2 * i + b
                        pltpu.make_async_copy(drain, rows.at[b],
                                              sems[b]).wait()
                        pltpu.sync_copy(rows.at[b], slab.at[idx_d.at[j]],
                                        add=True)

                        @pl.when(j + 2 < IB)
                        def _():
                            pltpu.async_copy(tbl.at[idx_s.at[j + 2]],
                                             rows.at[b], sems[b])
                    return c2

                lax.fori_loop(0, IB // 2, body, 0)
                return carry

            lax.fori_loop(0, nblk, blk_body, 0)
            plsc.subcore_barrier()
            pltpu.sync_copy(slab.at[pl.ds(sid * FR, FR)],
                            out.at[outidx].at[pl.ds(sid * FR, FR)])
            if k + 1 < chunks_per_core:
                plsc.subcore_barrier()

    return agg


def _mm(a, w):
    return lax.dot_general(a, w, (((1,), (1,)), ((), ())),
                           precision=lax.Precision.HIGHEST,
                           preferred_element_type=jnp.float32)


def _ln(x, g, b):
    mu = jnp.mean(x, axis=-1, keepdims=True)
    var = jnp.mean((x - mu) ** 2, axis=-1, keepdims=True)
    return (x - mu) / jnp.sqrt(var + EPS) * g + b


def _gelu(x):
    return 0.5 * x * (1.0 + lax.erf(x * 0.7071067811865476))


def _row_spec(d):
    return pl.BlockSpec((1, d), lambda i: (0, 0))


def _full2_spec(shape):
    return pl.BlockSpec(shape, lambda i: (0, 0))


def _tile_spec(d):
    return pl.BlockSpec((TN, d), lambda i: (i, 0))


def _chunk_spec(c):
    return pl.BlockSpec((c, TN, 128), lambda i: (0, i, 0))


def _ln_chunks(ys, g_ref, be_ref, width):
    m = ys[0].sum(axis=1, keepdims=True)
    for y in ys[1:]:
        m = m + y.sum(axis=1, keepdims=True)
    m = m / width
    v = ((ys[0] - m) ** 2).sum(axis=1, keepdims=True)
    for y in ys[1:]:
        v = v + ((y - m) ** 2).sum(axis=1, keepdims=True)
    v = v / width
    inv = 1.0 / jnp.sqrt(v + EPS)
    outs = []
    for c, y in enumerate(ys):
        g = g_ref[:, c * 128:(c + 1) * 128]
        b = be_ref[:, c * 128:(c + 1) * 128]
        outs.append((y - m) * inv * g + b)
    return outs


def _dinv_body(deg_ref, out_ref):
    d = 1.0 + deg_ref[0, :, 0:1] + deg_ref[1, :, 0:1]
    out_ref[...] = lax.rsqrt(d)


def _dinv(deg):
    return pl.pallas_call(
        _dinv_body,
        grid=(NT,),
        in_specs=[pl.BlockSpec((2, TN, 16), lambda i: (0, i, 0))],
        out_specs=_tile_spec(1),
        out_shape=jax.ShapeDtypeStruct((NPAD, 1), jnp.float32),
    )(deg)


def _s0a_body(x_ref, win_ref, bin_ref, gin_ref, bein_ref, h_ref):
    h = _mm(x_ref[...], win_ref[...]) + bin_ref[...]
    h_ref[...] = _gelu(_ln(h, gin_ref[...], bein_ref[...]))


def _s0a(xp, W_in, b_in, g_in, be_in):
    return pl.pallas_call(
        _s0a_body,
        grid=(NT,),
        in_specs=[
            _tile_spec(128),
            _full2_spec((512, 128)),
            _row_spec(512), _row_spec(512), _row_spec(512),
        ],
        out_specs=_tile_spec(512),
        out_shape=jax.ShapeDtypeStruct((NPAD, 512), jnp.float32),
    )(xp, W_in, b_in, g_in, be_in)


def _s0b_body(h_ref, wc1_ref, dinv_ref, tbl_ref):
    h = h_ref[...]
    dv = dinv_ref[...]
    for d in range(4):
        w = wc1_ref[...][d * 128:(d + 1) * 128, :]
        tbl_ref[d] = _mm(h, w) * dv


def _s0b(h0, Wc1, dinv):
    return pl.pallas_call(
        _s0b_body,
        grid=(NT,),
        in_specs=[
            _tile_spec(512),
            _full2_spec((512, 512)),
            _tile_spec(1),
        ],
        out_specs=_chunk_spec(4),
        out_shape=jax.ShapeDtypeStruct((4, NPAD, 128), jnp.float32),
    )(h0, Wc1, dinv)


def _make_t2(C, C2, partial_in):
    width = C * 128
    n_in = 2 if partial_in else C

    def body(agg_ref, tbl_ref, dinv_ref, b_ref, g_ref, be_ref, w2_ref,
             out_ref):
        dv = dinv_ref[...]
        ys = []
        for c in range(C):
            if partial_in:
                a = agg_ref[0] + agg_ref[1]
            else:
                a = agg_ref[c]
            b = b_ref[:, c * 128:(c + 1) * 128]
            ys.append(dv * (a + tbl_ref[c]) + b)
        zs = [_gelu(z) for z in _ln_chunks(ys, g_ref, be_ref, width)]
        w2 = w2_ref[...]
        for d in range(C2):
            acc = None
            for c in range(C):
                wblk = w2[d * 128:(d + 1) * 128, c * 128:(c + 1) * 128]
                t = _mm(zs[c], wblk)
                acc = t if acc is None else acc + t
            out_ref[d] = acc * dv

    def call(agg, tbl, dinv, b, g, be, W2):
        return pl.pallas_call(
            body,
            grid=(NT,),
            in_specs=[
                _chunk_spec(n_in), _chunk_spec(C), _tile_spec(1),
                _row_spec(width), _row_spec(width), _row_spec(width),
                _full2_spec((C2 * 128, width)),
            ],
            out_specs=_chunk_spec(C2),
            out_shape=jax.ShapeDtypeStruct((C2, NPAD, 128), jnp.float32),
        )(agg, tbl, dinv, b, g, be, W2)

    return call


def _make_t3(C, C_next, skip_width, partial_in):
    width = C * 128
    n_in = 2 if partial_in else C
    proj = skip_width > 0

    def body(*refs):
        it = iter(refs)
        agg_ref = next(it); tbl_ref = next(it); dinv_ref = next(it)
        b_ref = next(it); g_ref = next(it); be_ref = next(it)
        hprev_ref = next(it)
        if proj:
            ws_ref = next(it); bs_ref = next(it)
        wn_ref = next(it)
        hout_ref = next(it); tbl_out_ref = next(it)

        dv = dinv_ref[...]
        ys = []
        for c in range(C):
            a = (agg_ref[0] + agg_ref[1]) if partial_in else agg_ref[c]
            b = b_ref[:, c * 128:(c + 1) * 128]
            ys.append(dv * (a + tbl_ref[c]) + b)
        os_ = _ln_chunks(ys, g_ref, be_ref, width)
        hp = hprev_ref[...]
        hs = []
        for c in range(C):
            if proj:
                wblk = ws_ref[...][c * 128:(c + 1) * 128, :]
                skip = _mm(hp, wblk) + bs_ref[:, c * 128:(c + 1) * 128]
            else:
                skip = hp[:, c * 128:(c + 1) * 128]
            h = _gelu(os_[c] + skip)
            hs.append(h)
            hout_ref[:, c * 128:(c + 1) * 128] = h
        wn = wn_ref[...]
        for d in range(C_next):
            acc = None
            for c in range(C):
                wblk = wn[d * 128:(d + 1) * 128, c * 128:(c + 1) * 128]
                t = _mm(hs[c], wblk)
                acc = t if acc is None else acc + t
            tbl_out_ref[d] = acc * dv

    def call(agg, tbl, dinv, b, g, be, hprev, ws, bs, Wn):
        in_specs = [
            _chunk_spec(n_in), _chunk_spec(C), _tile_spec(1),
            _row_spec(width), _row_spec(width), _row_spec(width),
            _tile_spec(skip_width if proj else width),
        ]
        args = [agg, tbl, dinv, b, g, be, hprev]
        if proj:
            in_specs += [_full2_spec((width, skip_width)), _row_spec(width)]
            args += [ws, bs]
        in_specs.append(_full2_spec((C_next * 128, width)))
        args.append(Wn)
        return pl.pallas_call(
            body,
            grid=(NT,),
            in_specs=in_specs,
            out_specs=[_tile_spec(width), _chunk_spec(C_next)],
            out_shape=[
                jax.ShapeDtypeStruct((NPAD, width), jnp.float32),
                jax.ShapeDtypeStruct((C_next, NPAD, 128), jnp.float32),
            ],
        )(*args)

    return call


def _t3_final_body(agg_ref, tbl_ref, dinv_ref, b_ref, g_ref, be_ref,
                   hprev_ref, ws_ref, bs_ref,
                   we1_ref, be1_ref, ge1_ref, bee1_ref,
                   we2_ref, be2_ref, ge2_ref, bee2_ref,
                   pooled_ref):
    i = pl.program_id(0)
    dv = dinv_ref[...]
    y = dv * (agg_ref[0] + agg_ref[1] + tbl_ref[0]) + b_ref[...]
    o = _ln(y, g_ref[...], be_ref[...])
    skip = _mm(hprev_ref[...], ws_ref[...]) + bs_ref[...]
    h3 = _gelu(o + skip)
    e = _mm(h3, we1_ref[...]) + be1_ref[...]
    e = _gelu(_ln(e, ge1_ref[...], bee1_ref[...]))
    node = _mm(e, we2_ref[...]) + be2_ref[...]
    node = _ln(node, ge2_ref[...], bee2_ref[...])
    ridx = i * TN + lax.broadcasted_iota(jnp.int32, (TN, 1), 0)
    mask = ridx < N
    nsum = jnp.sum(jnp.where(mask, node, 0.0), axis=0, keepdims=True)
    nmax = jnp.max(jnp.where(mask, node, -jnp.inf), axis=0, keepdims=True)

    @pl.when(i == 0)
    def _():
        pooled_ref[0:1] = nsum
        pooled_ref[1:2] = nmax

    @pl.when(i > 0)
    def _():
        pooled_ref[0:1] += nsum
        pooled_ref[1:2] = jnp.maximum(pooled_ref[1:2], nmax)


def _t3_final(agg, tbl, dinv, b, g, be, hprev, ws, bs,
              We1, be1, ge1, bee1, We2, be2, ge2, bee2):
    return pl.pallas_call(
        _t3_final_body,
        grid=(NT,),
        in_specs=[
            _chunk_spec(2), _chunk_spec(1), _tile_spec(1),
            _row_spec(128), _row_spec(128), _row_spec(128),
            _tile_spec(256),
            _full2_spec((128, 256)), _row_spec(128),
            _full2_spec((256, 128)), _row_spec(256), _row_spec(256),
            _row_spec(256),
            _full2_spec((128, 256)), _row_spec(128), _row_spec(128),
            _row_spec(128),
        ],
        out_specs=pl.BlockSpec((8, 128), lambda i: (0, 0)),
        out_shape=jax.ShapeDtypeStruct((8, 128), jnp.float32),
    )(agg, tbl, dinv, b, g, be, hprev, ws, bs,
      We1, be1, ge1, bee1, We2, be2, ge2, bee2)


def _head_body(pooled_ref,
               cw1_ref, cb1_ref, cg1_ref, cbb1_ref, cw2_ref, cb2_ref,
               cg2_ref, cbb2_ref, cw3_ref, cb3_ref,
               rw1_ref, rb1_ref, rg1_ref, rbb1_ref, rw2_ref, rb2_ref,
               rg2_ref, rbb2_ref, rw3_ref, rb3_ref,
               cls_ref, reg_ref):
    s = pooled_ref[0:1]
    mx = pooled_ref[1:2]
    g = jnp.concatenate([s / float(N), mx, s], axis=1)

    def mlp(w1, b1, g1, bb1, w2, b2, g2, bb2, w3, b3):
        z = _mm(g, w1[...]) + b1[...]
        z = _gelu(_ln(z, g1[...], bb1[...]))
        z = _mm(z, w2[...]) + b2[...]
        z = _gelu(_ln(z, g2[...], bb2[...]))
        return _mm(z, w3[...]) + b3[...]

    cls_ref[...] = mlp(cw1_ref, cb1_ref, cg1_ref, cbb1_ref, cw2_ref,
                       cb2_ref, cg2_ref, cbb2_ref, cw3_ref, cb3_ref)
    reg_ref[...] = jax.nn.sigmoid(
        mlp(rw1_ref, rb1_ref, rg1_ref, rbb1_ref, rw2_ref, rb2_ref,
            rg2_ref, rbb2_ref, rw3_ref, rb3_ref))


def _pad_head_w3(w3, b3):
    od = w3.shape[0]
    return (jnp.pad(w3, ((0, 128 - od), (0, 0))),
            jnp.pad(b3.reshape(1, -1), ((0, 0), (0, 128 - od))))


def _head(pooled, cparams, rparams):
    def spec(a):
        return pl.BlockSpec(a.shape, lambda: tuple(0 for _ in a.shape))

    args = [pooled] + list(cparams) + list(rparams)
    return pl.pallas_call(
        _head_body,
        in_specs=[spec(a) for a in args],
        out_specs=[pl.BlockSpec((1, 128), lambda: (0, 0)),
                   pl.BlockSpec((1, 128), lambda: (0, 0))],
        out_shape=[jax.ShapeDtypeStruct((1, 128), jnp.float32),
                   jax.ShapeDtypeStruct((1, 128), jnp.float32)],
    )(*args)


def kernel(x, edge_index, W_in, b_in, g_in, be_in,
           Wc1_0, bc1_0, Wc2_0, bc2_0, g1_0, be1_0, g2_0, be2_0,
           Wc1_1, bc1_1, Wc2_1, bc2_1, g1_1, be1_1, g2_1, be2_1, Ws_1, bs_1,
           Wc1_2, bc1_2, Wc2_2, bc2_2, g1_2, be1_2, g2_2, be2_2, Ws_2, bs_2,
           We1, be1, ge1, bee1, We2, be2, ge2, bee2,
           c_W1, c_b1, c_g1, c_bb1, c_W2, c_b2, c_g2, c_bb2, c_W3, c_b3,
           r_W1, r_b1, r_g1, r_bb1, r_W2, r_b2, r_g2, r_bb2, r_W3, r_b3):
    r = lambda a: a.reshape(1, -1)
    xp = jnp.pad(x, ((0, NPAD - N), (0, 0)))
    src = edge_index[0]
    dst = edge_index[1]
    srcp = jnp.concatenate(
        [src, jnp.zeros((EPAD - E,), jnp.int32)]).reshape(EPAD // 128, 128)
    dstp = jnp.concatenate(
        [dst, jnp.full((EPAD - E,), DUMMY_DST, jnp.int32)]
    ).reshape(EPAD // 128, 128)
    zeros16 = jnp.zeros((640, 16), jnp.float32)
    ones16 = jnp.ones((128, 16), jnp.float32)
    zeros128 = jnp.zeros((640, 128), jnp.float32)

    deg = _get_sc_degree()(dstp, ones16, zeros16)
    h0 = _s0a(xp, W_in, r(b_in), r(g_in), r(be_in))
    dinv = _dinv(deg)
    tbl = _s0b(h0, Wc1_0, dinv)

    t2_0 = _make_t2(4, 4, False)
    t3_0 = _make_t3(4, 2, 0, False)
    t2_1 = _make_t2(2, 2, False)
    t3_1 = _make_t3(2, 1, 512, False)
    t2_2 = _make_t2(1, 1, True)

    agg = _make_sc_agg(4)(tbl, srcp, dstp, zeros128)
    tbl = t2_0(agg, tbl, dinv, r(bc1_0), r(g1_0), r(be1_0), Wc2_0)
    agg = _make_sc_agg(4)(tbl, srcp, dstp, zeros128)
    h1, tbl = t3_0(agg, tbl, dinv, r(bc2_0), r(g2_0), r(be2_0), h0,
                   None, None, Wc1_1)

    agg = _make_sc_agg(2)(tbl, srcp, dstp, zeros128)
    tbl = t2_1(agg, tbl, dinv, r(bc1_1), r(g1_1), r(be1_1), Wc2_1)
    agg = _make_sc_agg(2)(tbl, srcp, dstp, zeros128)
    h2, tbl = t3_1(agg, tbl, dinv, r(bc2_1), r(g2_1), r(be2_1), h1,
                   Ws_1, r(bs_1), Wc1_2)

    agg = _make_sc_agg(1)(tbl, srcp, dstp, zeros128)
    tbl = t2_2(agg, tbl, dinv, r(bc1_2), r(g1_2), r(be1_2), Wc2_2)
    agg = _make_sc_agg(1)(tbl, srcp, dstp, zeros128)
    pooled = _t3_final(agg, tbl, dinv, r(bc2_2), r(g2_2), r(be2_2), h2,
                       Ws_2, r(bs_2),
                       We1, r(be1), r(ge1), r(bee1),
                       We2, r(be2), r(ge2), r(bee2))

    cls, reg = _head(
        pooled,
        (c_W1, r(c_b1), r(c_g1), r(c_bb1), c_W2, r(c_b2), r(c_g2),
         r(c_bb2), *_pad_head_w3(c_W3, c_b3)),
        (r_W1, r(r_b1), r(r_g1), r(r_bb1), r_W2, r(r_b2), r(r_g2),
         r(r_bb2), *_pad_head_w3(r_W3, r_b3)))
    return (cls[:, :5], reg[:, :1])
